# baseline (speedup 1.0000x reference)
import functools

import jax
import jax.numpy as jnp
from jax import lax
from jax.experimental import pallas as pl
from jax.experimental.pallas import tpu as pltpu
from jax.experimental.pallas import tpu_sc as plsc

NC = 2
NS = 16
NW = NC * NS
IDX_W = 128


def _gather_call(batch, seq, dim, nb, nbuf):
    per_worker = batch // NW
    n_chunks = per_worker // nb
    assert batch % NW == 0 and per_worker % nb == 0 and n_chunks % nbuf == 0
    splits = []
    t0 = 0
    while t0 < seq:
        w = min(IDX_W, seq - t0)
        splits.append((t0, w))
        t0 += w

    mesh = plsc.VectorSubcoreMesh(
        core_axis_name="c", subcore_axis_name="s",
        num_cores=NC, num_subcores=NS,
    )

    @functools.partial(
        pl.kernel,
        out_type=jax.ShapeDtypeStruct((batch, seq, dim), jnp.float32),
        mesh=mesh,
        compiler_params=pltpu.CompilerParams(use_tc_tiling_on_sc=False),
        scratch_types=[
            pltpu.VMEM((per_worker, seq), jnp.int32),
            [pltpu.VMEM((nb, seq, dim), jnp.float32) for _ in range(nbuf)],
            [pltpu.SemaphoreType.DMA for _ in range(nbuf)],
            [pltpu.SemaphoreType.DMA for _ in range(nbuf)],
        ],
    )
    def gather_kernel(ids_hbm, table_hbm, out_hbm, idx_v, rows_v, gsems, osems):
        wid = lax.axis_index("s") * NC + lax.axis_index("c")
        seq_base = wid * per_worker

        pltpu.sync_copy(ids_hbm.at[pl.ds(seq_base, per_worker)], idx_v)

        def gather_copies(c, b):
            cps = []
            for i in range(nb):
                for (t0, w) in splits:
                    cps.append(pltpu.make_async_copy(
                        table_hbm.at[idx_v.at[c * nb + i, pl.ds(t0, w)]],
                        rows_v[b].at[i, pl.ds(t0, w)],
                        gsems[b],
                    ))
            return cps

        for b in range(nbuf):
            for cp in gather_copies(b, b):
                cp.start()

        def slot(c, b):
            for cp in gather_copies(c, b):
                cp.wait()
            wb = pltpu.make_async_copy(
                rows_v[b], out_hbm.at[pl.ds(seq_base + c * nb, nb)],
                osems[b])
            wb.start()
            nxt = c + nbuf

            @pl.when(nxt < n_chunks)
            def _():
                wb.wait()
                for cp in gather_copies(nxt, b):
                    cp.start()

        def body(g, carry):
            for b in range(nbuf):
                slot(g * nbuf + b, b)
            return carry

        lax.fori_loop(0, n_chunks // nbuf, body, 0, unroll=False)
        for b in range(nbuf):
            pltpu.make_async_copy(
                rows_v[b], out_hbm.at[pl.ds(seq_base, nb)], osems[b]
            ).wait()

    return gather_kernel


def kernel(token_ids, embeddings):
    batch, seq = token_ids.shape
    vocab, dim = embeddings.shape

    x = _gather_call(batch, seq, dim, 4, 2)(token_ids, embeddings)

    mask = jnp.ones((batch, seq), dtype=jnp.float32)
    time_steps = jnp.array(seq, dtype=jnp.int32)
    return (x, mask, time_steps)
